# baseline (speedup 1.0000x reference)
import functools

import jax
import jax.numpy as jnp
from jax import lax
from jax.experimental import pallas as pl
from jax.experimental.pallas import tpu as pltpu
from jax.experimental.pallas import tpu_sc as plsc

VOCAB = 1000000
EMB_DIM = 64
BATCH = 4096
HIST = 200

_INFO = plsc.get_sparse_core_info()
_NC = _INFO.num_cores
_NS = _INFO.num_subcores
_NW = _NC * _NS

_R = BATCH * HIST
_R_PER_W = _R // _NW
_CHUNK = 128
_NCHUNK = _R_PER_W // _CHUNK

_mesh = plsc.VectorSubcoreMesh(core_axis_name="c", subcore_axis_name="s")


@functools.partial(
    pl.kernel,
    mesh=_mesh,
    out_type=jax.ShapeDtypeStruct((_R, EMB_DIM), jnp.float32),
    scratch_types=[
        pltpu.VMEM((_NCHUNK, _CHUNK), jnp.int32),
        pltpu.VMEM((_CHUNK, EMB_DIM), jnp.float32),
        pltpu.SemaphoreType.DMA,
    ],
    compiler_params=pltpu.CompilerParams(use_tc_tiling_on_sc=False),
)
def _sc_gather(table_hbm, idx_hbm, out_hbm, idx_v, rows_v, sem):
    wid = lax.axis_index("s") * _NC + lax.axis_index("c")
    base = wid * _R_PER_W
    pltpu.sync_copy(idx_hbm.at[wid], idx_v)

    def body(j, _):
        pltpu.async_copy(table_hbm.at[idx_v.at[j]], rows_v, sem).wait()
        pltpu.sync_copy(rows_v, out_hbm.at[pl.ds(base + j * _CHUNK, _CHUNK)])
        return _

    lax.fori_loop(0, _NCHUNK, body, None)


def kernel(x, weight):
    idx = x.astype(jnp.int32).reshape(_NW, _NCHUNK, _CHUNK)
    out = _sc_gather(weight, idx)
    return out.reshape(BATCH, HIST, EMB_DIM)


# speedup vs baseline: 1.1119x; 1.1119x over previous
import functools

import jax
import jax.numpy as jnp
from jax import lax
from jax.experimental import pallas as pl
from jax.experimental.pallas import tpu as pltpu
from jax.experimental.pallas import tpu_sc as plsc

VOCAB = 1000000
EMB_DIM = 64
BATCH = 4096
HIST = 200

_INFO = plsc.get_sparse_core_info()
_NC = _INFO.num_cores
_NS = _INFO.num_subcores
_NW = _NC * _NS

_R = BATCH * HIST
_R_PER_W = _R // _NW
_CHUNK = 128
_NCHUNK = _R_PER_W // _CHUNK

_NBUF = 4
_NROUND = _NCHUNK // _NBUF

_mesh = plsc.VectorSubcoreMesh(core_axis_name="c", subcore_axis_name="s")


@functools.partial(
    pl.kernel,
    mesh=_mesh,
    out_type=jax.ShapeDtypeStruct((_R, EMB_DIM), jnp.float32),
    scratch_types=[
        pltpu.VMEM((_NCHUNK, _CHUNK), jnp.int32),
        pltpu.VMEM((_NBUF, _CHUNK, EMB_DIM), jnp.float32),
        pltpu.SemaphoreType.DMA((_NBUF,)),
        pltpu.SemaphoreType.DMA((_NBUF,)),
    ],
    compiler_params=pltpu.CompilerParams(use_tc_tiling_on_sc=False),
)
def _sc_gather(table_hbm, idx_hbm, out_hbm, idx_v, rows_v, gsem, ssem):
    wid = lax.axis_index("s") * _NC + lax.axis_index("c")
    base = wid * _R_PER_W
    pltpu.sync_copy(idx_hbm.at[wid], idx_v)

    def fire_gather(j, b):
        pltpu.async_copy(table_hbm.at[idx_v.at[j]], rows_v.at[b], gsem.at[b])

    def wait_gather(b):
        pltpu.make_async_copy(
            table_hbm.at[idx_v.at[0]], rows_v.at[b], gsem.at[b]).wait()

    def fire_store(j, b):
        pltpu.async_copy(
            rows_v.at[b], out_hbm.at[pl.ds(base + j * _CHUNK, _CHUNK)],
            ssem.at[b])

    def wait_store(b):
        pltpu.make_async_copy(
            rows_v.at[b], out_hbm.at[pl.ds(base, _CHUNK)], ssem.at[b]).wait()

    for b in range(_NBUF):
        fire_gather(b, b)

    def round_body(g, _):
        j0 = g * _NBUF
        for b in range(_NBUF):
            j = j0 + b
            wait_gather(b)
            fire_store(j, b)
            wait_store(b)
            fire_gather(j + _NBUF, b)
        return _

    lax.fori_loop(0, _NROUND - 1, round_body, None)

    j0 = (_NROUND - 1) * _NBUF
    for b in range(_NBUF):
        wait_gather(b)
        fire_store(j0 + b, b)
        wait_store(b)


def kernel(x, weight):
    idx = x.astype(jnp.int32).reshape(_NW, _NCHUNK, _CHUNK)
    out = _sc_gather(weight, idx)
    return out.reshape(BATCH, HIST, EMB_DIM)


# speedup vs baseline: 1.1146x; 1.0024x over previous
import functools

import jax
import jax.numpy as jnp
from jax import lax
from jax.experimental import pallas as pl
from jax.experimental.pallas import tpu as pltpu
from jax.experimental.pallas import tpu_sc as plsc

VOCAB = 1000000
EMB_DIM = 64
BATCH = 4096
HIST = 200

_INFO = plsc.get_sparse_core_info()
_NC = _INFO.num_cores
_NS = _INFO.num_subcores
_NW = _NC * _NS

_R = BATCH * HIST
_R_PER_W = _R // _NW
_CHUNK = 128
_NCHUNK = _R_PER_W // _CHUNK

_NBUF = 8
_LOOK = 4
_NROUND = _NCHUNK // _NBUF

_mesh = plsc.VectorSubcoreMesh(core_axis_name="c", subcore_axis_name="s")


@functools.partial(
    pl.kernel,
    mesh=_mesh,
    out_type=jax.ShapeDtypeStruct((_R, EMB_DIM), jnp.float32),
    scratch_types=[
        pltpu.VMEM((_NCHUNK, _CHUNK), jnp.int32),
        pltpu.VMEM((_NBUF, _CHUNK, EMB_DIM), jnp.float32),
        pltpu.SemaphoreType.DMA((_NBUF,)),
        pltpu.SemaphoreType.DMA((_NBUF,)),
    ],
    compiler_params=pltpu.CompilerParams(use_tc_tiling_on_sc=False),
)
def _sc_gather(table_hbm, idx_hbm, out_hbm, idx_v, rows_v, gsem, ssem):
    wid = lax.axis_index("s") * _NC + lax.axis_index("c")
    base = wid * _R_PER_W
    pltpu.sync_copy(idx_hbm.at[wid], idx_v)

    def fire_gather(j, b):
        pltpu.async_copy(table_hbm.at[idx_v.at[j]], rows_v.at[b], gsem.at[b])

    def wait_gather(b):
        pltpu.make_async_copy(
            table_hbm.at[idx_v.at[0]], rows_v.at[b], gsem.at[b]).wait()

    def fire_store(j, b):
        pltpu.async_copy(
            rows_v.at[b], out_hbm.at[pl.ds(base + j * _CHUNK, _CHUNK)],
            ssem.at[b])

    def wait_store(b):
        pltpu.make_async_copy(
            rows_v.at[b], out_hbm.at[pl.ds(base, _CHUNK)], ssem.at[b]).wait()

    for b in range(_LOOK):
        fire_gather(b, b)

    for j in range(_NBUF):
        b = j % _NBUF
        wait_gather(b)
        fire_store(j, b)
        bn = (j + _LOOK) % _NBUF
        if j >= _LOOK:
            wait_store(bn)
        fire_gather(j + _LOOK, bn)

    def round_body(g, _):
        j0 = g * _NBUF
        for b in range(_NBUF):
            j = j0 + b
            wait_gather(b)
            fire_store(j, b)
            bn = (j + _LOOK) % _NBUF
            wait_store(bn)
            fire_gather(j + _LOOK, bn)
        return _

    lax.fori_loop(1, _NROUND - 1, round_body, None)

    j0 = (_NROUND - 1) * _NBUF
    for b in range(_NBUF):
        j = j0 + b
        wait_gather(b)
        fire_store(j, b)
        bn = (j + _LOOK) % _NBUF
        wait_store(bn)
        if j + _LOOK < _NCHUNK:
            fire_gather(j + _LOOK, bn)
    for b in range(_LOOK):
        wait_store((j0 + _NBUF - _LOOK + b) % _NBUF)


def kernel(x, weight):
    idx = x.astype(jnp.int32).reshape(_NW, _NCHUNK, _CHUNK)
    out = _sc_gather(weight, idx)
    return out.reshape(BATCH, HIST, EMB_DIM)


# speedup vs baseline: 1.1148x; 1.0001x over previous
import functools

import jax
import jax.numpy as jnp
from jax import lax
from jax.experimental import pallas as pl
from jax.experimental.pallas import tpu as pltpu
from jax.experimental.pallas import tpu_sc as plsc

VOCAB = 1000000
EMB_DIM = 64
BATCH = 4096
HIST = 200

_INFO = plsc.get_sparse_core_info()
_NC = _INFO.num_cores
_NS = _INFO.num_subcores
_NW = _NC * _NS

_B_PER_W = BATCH // _NW
_SPLIT = 104
_SIZES = (_SPLIT, HIST - _SPLIT)
_NCHUNK = 2 * _B_PER_W
_NBUF = 8
_LOOK = 4
_NROUND = _NCHUNK // _NBUF

_mesh = plsc.VectorSubcoreMesh(core_axis_name="c", subcore_axis_name="s")


@functools.partial(
    pl.kernel,
    mesh=_mesh,
    out_type=jax.ShapeDtypeStruct((BATCH, HIST, EMB_DIM), jnp.float32),
    scratch_types=[
        pltpu.VMEM((_B_PER_W, HIST), jnp.int32),
        pltpu.VMEM((_NBUF, _SPLIT, EMB_DIM), jnp.float32),
        pltpu.SemaphoreType.DMA((_NBUF,)),
        pltpu.SemaphoreType.DMA((_NBUF,)),
    ],
    compiler_params=pltpu.CompilerParams(use_tc_tiling_on_sc=False),
)
def _sc_gather(table_hbm, idx_hbm, out_hbm, idx_v, rows_v, gsem, ssem):
    wid = lax.axis_index("s") * _NC + lax.axis_index("c")
    row0 = wid * _B_PER_W
    pltpu.sync_copy(idx_hbm.at[pl.ds(row0, _B_PER_W)], idx_v)

    def fire_gather(g, b):
        row = g * (_NBUF // 2) + b // 2
        half = b % 2
        size = _SIZES[half]
        idx = idx_v.at[row, pl.ds(half * _SPLIT, size)]
        pltpu.async_copy(
            table_hbm.at[idx], rows_v.at[b, pl.ds(0, size)], gsem.at[b])

    def wait_gather(b):
        size = _SIZES[b % 2]
        pltpu.make_async_copy(
            table_hbm.at[idx_v.at[0, pl.ds(0, size)]],
            rows_v.at[b, pl.ds(0, size)], gsem.at[b]).wait()

    def fire_store(g, b):
        row = g * (_NBUF // 2) + b // 2
        half = b % 2
        size = _SIZES[half]
        pltpu.async_copy(
            rows_v.at[b, pl.ds(0, size)],
            out_hbm.at[row0 + row, pl.ds(half * _SPLIT, size)], ssem.at[b])

    def wait_store(b):
        size = _SIZES[b % 2]
        pltpu.make_async_copy(
            rows_v.at[b, pl.ds(0, size)],
            out_hbm.at[row0, pl.ds((b % 2) * _SPLIT, size)],
            ssem.at[b]).wait()

    for b in range(_LOOK):
        fire_gather(0, b)

    for b in range(_NBUF):
        wait_gather(b)
        fire_store(0, b)
        bn = (b + _LOOK) % _NBUF
        if b >= _LOOK:
            wait_store(bn)
        fire_gather((b + _LOOK) // _NBUF, (b + _LOOK) % _NBUF)

    def round_body(g, _):
        for b in range(_NBUF):
            wait_gather(b)
            fire_store(g, b)
            bn = (b + _LOOK) % _NBUF
            wait_store(bn)
            gq, bq = divmod(b + _LOOK, _NBUF)
            fire_gather(g + gq, bq)
        return _

    lax.fori_loop(1, _NROUND - 1, round_body, None)

    g = _NROUND - 1
    for b in range(_NBUF):
        wait_gather(b)
        fire_store(g, b)
        bn = (b + _LOOK) % _NBUF
        wait_store(bn)
        if b + _LOOK < _NBUF:
            gq, bq = divmod(b + _LOOK, _NBUF)
            fire_gather(g + gq, bq)
    for b in range(_LOOK):
        wait_store((_NBUF - _LOOK + b) % _NBUF)


def kernel(x, weight):
    return _sc_gather(weight, x.astype(jnp.int32))
